# Initial kernel scaffold; baseline (speedup 1.0000x reference)
import jax
import jax.numpy as jnp
from jax.experimental import pallas as pl


def kernel(x, W_enc, b_enc, W_dec, b_dec):
    raise NotImplementedError("write your pallas kernel here")


# speedup vs baseline: 8.2470x; 8.2470x over previous
import functools

import jax
import jax.numpy as jnp
from jax import lax
from jax.experimental import pallas as pl
from jax.experimental.pallas import tpu as pltpu

K_TOP = 30


def _encode_body(x_ref, w_ref, benc_ref, bdec_ref, pre_ref):
    xc = x_ref[...] - bdec_ref[...]
    acc = lax.dot_general(
        xc, w_ref[...], (((1,), (1,)), ((), ())),
        preferred_element_type=jnp.float32,
    )
    pre_ref[...] = acc + benc_ref[...]


def _thresh_body(pre_ref, t_ref, *, k):
    p = pre_ref[...]

    def step(_, t):
        return jnp.max(jnp.where(p < t, p, -jnp.inf), axis=1, keepdims=True)

    t0 = jnp.full((p.shape[0], 1), jnp.inf, dtype=jnp.float32)
    t_ref[...] = lax.fori_loop(0, k, step, t0)


def _decode_body(pre_ref, t_ref, wdt_ref, bdec_ref, sparse_ref, recon_ref):
    j = pl.program_id(1)
    p = pre_ref[...]
    s = jnp.where(p >= t_ref[...], jnp.maximum(p, 0.0), 0.0)
    sparse_ref[...] = s
    contrib = lax.dot_general(
        s, wdt_ref[...], (((1,), (0,)), ((), ())),
        preferred_element_type=jnp.float32,
    )

    @pl.when(j == 0)
    def _():
        recon_ref[...] = contrib + bdec_ref[...]

    @pl.when(j != 0)
    def _():
        recon_ref[...] += contrib


@jax.jit
def kernel(x, W_enc, b_enc, W_dec, b_dec):
    B, D = x.shape
    F = W_enc.shape[0]
    BR = 512
    BF = 1024
    BR2 = 256

    benc2 = b_enc.reshape(1, F)
    bdec2 = b_dec.reshape(1, D)
    W_decT = W_dec.T

    pre = pl.pallas_call(
        _encode_body,
        grid=(B // BR, F // BF),
        in_specs=[
            pl.BlockSpec((BR, D), lambda i, j: (i, 0)),
            pl.BlockSpec((BF, D), lambda i, j: (j, 0)),
            pl.BlockSpec((1, BF), lambda i, j: (0, j)),
            pl.BlockSpec((1, D), lambda i, j: (0, 0)),
        ],
        out_specs=pl.BlockSpec((BR, BF), lambda i, j: (i, j)),
        out_shape=jax.ShapeDtypeStruct((B, F), jnp.float32),
        compiler_params=pltpu.CompilerParams(
            dimension_semantics=("parallel", "parallel"),
        ),
    )(x, W_enc, benc2, bdec2)

    t = pl.pallas_call(
        functools.partial(_thresh_body, k=K_TOP),
        grid=(B // BR2,),
        in_specs=[pl.BlockSpec((BR2, F), lambda i: (i, 0))],
        out_specs=pl.BlockSpec((BR2, 1), lambda i: (i, 0)),
        out_shape=jax.ShapeDtypeStruct((B, 1), jnp.float32),
        compiler_params=pltpu.CompilerParams(
            dimension_semantics=("parallel",),
        ),
    )(pre)

    sparse, recon = pl.pallas_call(
        _decode_body,
        grid=(B // BR, F // BF),
        in_specs=[
            pl.BlockSpec((BR, BF), lambda i, j: (i, j)),
            pl.BlockSpec((BR, 1), lambda i, j: (i, 0)),
            pl.BlockSpec((BF, D), lambda i, j: (j, 0)),
            pl.BlockSpec((1, D), lambda i, j: (0, 0)),
        ],
        out_specs=[
            pl.BlockSpec((BR, BF), lambda i, j: (i, j)),
            pl.BlockSpec((BR, D), lambda i, j: (i, 0)),
        ],
        out_shape=[
            jax.ShapeDtypeStruct((B, F), jnp.float32),
            jax.ShapeDtypeStruct((B, D), jnp.float32),
        ],
        compiler_params=pltpu.CompilerParams(
            dimension_semantics=("parallel", "arbitrary"),
        ),
    )(pre, t, W_decT, bdec2)

    return (recon, sparse)
